# baseline (speedup 1.0000x reference)
import functools

import jax
import jax.numpy as jnp
from jax import lax
from jax.experimental import pallas as pl
from jax.experimental.pallas import tpu as pltpu
from jax.experimental.pallas import tpu_sc as plsc

_CHUNK = 128
_PACK = 4


@functools.lru_cache(maxsize=None)
def _make_gather(coarse_rows: int, batch: int):
    info = plsc.get_sparse_core_info()
    nc, ns = info.num_cores, info.num_subcores
    nw = nc * ns
    b_per_w = batch // nw
    n_chunks = b_per_w // _CHUNK
    groups_per_chunk = _CHUNK // 16
    mesh = plsc.VectorSubcoreMesh(core_axis_name="c", subcore_axis_name="s")

    @functools.partial(
        pl.kernel,
        mesh=mesh,
        compiler_params=pltpu.CompilerParams(needs_layout_passes=False),
        out_type=jax.ShapeDtypeStruct((batch // _PACK, 128), jnp.float32),
        scratch_types=[
            pltpu.VMEM((n_chunks, _CHUNK), jnp.int32),
            pltpu.VMEM((n_chunks, _CHUNK), jnp.int32),
            pltpu.VMEM((b_per_w, 128), jnp.float32),
            pltpu.VMEM((b_per_w // _PACK, 128), jnp.float32),
            [pltpu.SemaphoreType.DMA] * n_chunks,
        ],
    )
    def gather_kernel(idx_hbm, table_hbm, out_hbm, idx_v, cidx_v, big_v,
                      out_v, sems):
        wid = lax.axis_index("s") * nc + lax.axis_index("c")
        pltpu.sync_copy(idx_hbm.at[pl.ds(wid * n_chunks, n_chunks)], idx_v)
        for j in range(n_chunks):
            for k in range(_CHUNK // 16):
                sl = pl.ds(k * 16, 16)
                cidx_v[j, sl] = lax.shift_right_logical(idx_v[j, sl], 2)
        copies = [
            pltpu.async_copy(
                table_hbm.at[cidx_v.at[j]],
                big_v.at[pl.ds(j * _CHUNK, _CHUNK)],
                sems[j],
            )
            for j in range(n_chunks)
        ]
        lanes = lax.iota(jnp.int32, 16)
        for j in range(n_chunks):
            copies[j].wait()

            def group_body(g, carry, j=j):
                row_vec = g * 16 + lanes
                iv = idx_v[j, pl.ds((g - groups_per_chunk * j) * 16, 16)]
                off = (iv & (_PACK - 1)) * 32
                qrow = lax.shift_right_logical(row_vec, 2)
                mbase = (row_vec & (_PACK - 1)) * 32
                for c in range(32):
                    vals = plsc.load_gather(big_v, [row_vec, off + c])
                    plsc.store_scatter(out_v, [qrow, mbase + c], vals)
                return carry

            lax.fori_loop(groups_per_chunk * j, groups_per_chunk * (j + 1),
                          group_body, 0)
        pltpu.sync_copy(
            out_v, out_hbm.at[pl.ds(wid * (b_per_w // _PACK), b_per_w // _PACK)]
        )

    return gather_kernel


def kernel(inputs, user_table):
    batch, = inputs.shape
    num_rows, embed_dim = user_table.shape
    idx2d = inputs.astype(jnp.int32).reshape(batch // _CHUNK, _CHUNK)
    table2 = user_table.reshape(num_rows * embed_dim // 128, 128)
    gather = _make_gather(table2.shape[0], batch)
    out = gather(idx2d, table2)
    return out.reshape(batch, embed_dim)


# speedup vs baseline: 3.9515x; 3.9515x over previous
import functools

import jax
import jax.numpy as jnp
from jax import lax
from jax.experimental import pallas as pl
from jax.experimental.pallas import tpu as pltpu
from jax.experimental.pallas import tpu_sc as plsc

_IDXROW = 128
_GRP = 8


@functools.lru_cache(maxsize=None)
def _make_gather(num_rows: int, embed_dim: int, batch: int):
    info = plsc.get_sparse_core_info()
    nc, ns = info.num_cores, info.num_subcores
    nw = nc * ns
    b_per_w = batch // nw
    n_rows = b_per_w // _IDXROW
    n_grps = b_per_w // _GRP
    mesh = plsc.VectorSubcoreMesh(core_axis_name="c", subcore_axis_name="s")

    @functools.partial(
        pl.kernel,
        mesh=mesh,
        compiler_params=pltpu.CompilerParams(needs_layout_passes=False),
        out_type=jax.ShapeDtypeStruct((embed_dim, batch), jnp.float32),
        scratch_types=[
            pltpu.VMEM((n_rows, _IDXROW), jnp.int32),
            pltpu.VMEM((2 * _GRP * embed_dim, 128), jnp.float32),
            pltpu.VMEM((embed_dim, b_per_w), jnp.float32),
            pltpu.SemaphoreType.DMA,
            pltpu.SemaphoreType.DMA,
        ],
    )
    def gather_kernel(idx_hbm, table_hbm, out_hbm, idx_v, ring_v, cols_v,
                      sem0, sem1):
        wid = lax.axis_index("s") * nc + lax.axis_index("c")
        pltpu.sync_copy(idx_hbm.at[pl.ds(wid * n_rows, n_rows)], idx_v)
        lanes = lax.iota(jnp.int32, 16)
        sems = (sem0, sem1)

        def load_vec(g):
            j = lax.shift_right_logical(g, 4)
            sl = lax.shift_right_logical(g & 15, 1) * 16
            return idx_v[j, pl.ds(sl, 16)]

        def fire(g, half, bank, vec):
            for l in range(_GRP * half, _GRP * half + _GRP):
                rt = lax.shift_right_logical(vec[l], 7)
                off = pl.multiple_of(rt * 128, 128)
                slot = bank * _GRP + (l - _GRP * half)
                pltpu.async_copy(
                    table_hbm.at[:, pl.ds(off, 128)],
                    ring_v.at[pl.ds(slot * embed_dim, embed_dim)],
                    sems[bank],
                )

        def drain(bank):
            for _ in range(_GRP):
                pltpu.make_async_copy(
                    table_hbm.at[:, pl.ds(0, 128)],
                    ring_v.at[pl.ds(0, embed_dim)],
                    sems[bank],
                ).wait()

        def extract(g, half, bank, vec):
            for l in range(_GRP * half, _GRP * half + _GRP):
                rloc = vec[l] & 127
                base = (bank * _GRP + (l - _GRP * half)) * embed_dim
                col = g * _GRP + (l - _GRP * half)
                for h in range(embed_dim // 16):
                    vals = plsc.load_gather(
                        ring_v, [base + h * 16 + lanes, rloc + 0 * lanes]
                    )
                    plsc.store_scatter(
                        cols_v, [h * 16 + lanes, col + 0 * lanes], vals
                    )

        vec0 = load_vec(0)
        fire(0, 0, 0, vec0)

        def body(hh, vec_e):
            g = 2 * hh
            vec_o = vec_e
            fire(g + 1, 1, 1, vec_o)
            drain(0)
            extract(g, 0, 0, vec_e)
            vec_e2 = load_vec(g + 2)
            fire(g + 2, 0, 0, vec_e2)
            drain(1)
            extract(g + 1, 1, 1, vec_o)
            return vec_e2

        vec_e = lax.fori_loop(0, n_grps // 2 - 1, body, vec0)
        g = n_grps - 2
        fire(g + 1, 1, 1, vec_e)
        drain(0)
        extract(g, 0, 0, vec_e)
        drain(1)
        extract(g + 1, 1, 1, vec_e)

        pltpu.sync_copy(cols_v, out_hbm.at[:, pl.ds(wid * b_per_w, b_per_w)])

    return gather_kernel


def kernel(inputs, user_table):
    batch, = inputs.shape
    num_rows, embed_dim = user_table.shape
    idx2d = inputs.astype(jnp.int32).reshape(batch // _IDXROW, _IDXROW)
    gather = _make_gather(num_rows, embed_dim, batch)
    out_t = gather(idx2d, user_table.T)
    return out_t.T


# speedup vs baseline: 4.0828x; 1.0332x over previous
import functools

import jax
import jax.numpy as jnp
from jax import lax
from jax.experimental import pallas as pl
from jax.experimental.pallas import tpu as pltpu
from jax.experimental.pallas import tpu_sc as plsc

_IDXROW = 128
_GRP = 4
_NBANK = 4


@functools.lru_cache(maxsize=None)
def _make_gather(num_rows: int, embed_dim: int, batch: int):
    info = plsc.get_sparse_core_info()
    nc, ns = info.num_cores, info.num_subcores
    nw = nc * ns
    b_per_w = batch // nw
    n_rows = b_per_w // _IDXROW
    n_grps = b_per_w // _GRP
    n_iters = n_grps // _GRP
    mesh = plsc.VectorSubcoreMesh(core_axis_name="c", subcore_axis_name="s")

    @functools.partial(
        pl.kernel,
        mesh=mesh,
        compiler_params=pltpu.CompilerParams(needs_layout_passes=False),
        out_type=jax.ShapeDtypeStruct((embed_dim, batch), jnp.float32),
        scratch_types=[
            pltpu.VMEM((n_rows, _IDXROW), jnp.int32),
            pltpu.VMEM((_NBANK * _GRP * embed_dim, 128), jnp.float32),
            pltpu.VMEM((embed_dim, b_per_w), jnp.float32),
            [pltpu.SemaphoreType.DMA] * _NBANK,
        ],
    )
    def gather_kernel(idx_hbm, table_hbm, out_hbm, idx_v, ring_v, cols_v,
                      sems):
        wid = lax.axis_index("s") * nc + lax.axis_index("c")
        pltpu.sync_copy(idx_hbm.at[pl.ds(wid * n_rows, n_rows)], idx_v)
        lanes = lax.iota(jnp.int32, 16)

        def load_vec(t):
            j = lax.shift_right_logical(t, 3)
            sl = (t & 7) * 16
            return idx_v[j, pl.ds(sl, 16)]

        def fire(vec, lanebase, bank):
            for l in range(_GRP):
                rt = lax.shift_right_logical(vec[lanebase + l], 7)
                off = pl.multiple_of(rt * 128, 128)
                slot = bank * _GRP + l
                pltpu.async_copy(
                    table_hbm.at[:, pl.ds(off, 128)],
                    ring_v.at[pl.ds(slot * embed_dim, embed_dim)],
                    sems[bank],
                )

        def drain(bank):
            for _ in range(_GRP):
                pltpu.make_async_copy(
                    table_hbm.at[:, pl.ds(0, 128)],
                    ring_v.at[pl.ds(0, embed_dim)],
                    sems[bank],
                ).wait()

        def extract(g, vec, lanebase, bank):
            for l in range(_GRP):
                rloc = vec[lanebase + l] & 127
                base = (bank * _GRP + l) * embed_dim
                col = g * _GRP + l
                for h in range(embed_dim // 16):
                    vals = plsc.load_gather(
                        ring_v, [base + h * 16 + lanes, rloc + 0 * lanes]
                    )
                    plsc.store_scatter(
                        cols_v, [h * 16 + lanes, col + 0 * lanes], vals
                    )

        vec0 = load_vec(0)
        fire(vec0, 0, 0)
        fire(vec0, _GRP, 1)
        fire(vec0, 2 * _GRP, 2)

        def body(it, vec_cur):
            g0 = it * _GRP
            vec_next = load_vec(it + 1)
            drain(0)
            extract(g0, vec_cur, 0, 0)
            fire(vec_cur, 3 * _GRP, 3)
            drain(1)
            extract(g0 + 1, vec_cur, _GRP, 1)
            fire(vec_next, 0, 0)
            drain(2)
            extract(g0 + 2, vec_cur, 2 * _GRP, 2)
            fire(vec_next, _GRP, 1)
            drain(3)
            extract(g0 + 3, vec_cur, 3 * _GRP, 3)
            fire(vec_next, 2 * _GRP, 2)
            return vec_next

        vec_cur = lax.fori_loop(0, n_iters - 1, body, vec0)
        g0 = (n_iters - 1) * _GRP
        drain(0)
        extract(g0, vec_cur, 0, 0)
        fire(vec_cur, 3 * _GRP, 3)
        drain(1)
        extract(g0 + 1, vec_cur, _GRP, 1)
        drain(2)
        extract(g0 + 2, vec_cur, 2 * _GRP, 2)
        drain(3)
        extract(g0 + 3, vec_cur, 3 * _GRP, 3)

        pltpu.sync_copy(cols_v, out_hbm.at[:, pl.ds(wid * b_per_w, b_per_w)])

    return gather_kernel


def kernel(inputs, user_table):
    batch, = inputs.shape
    num_rows, embed_dim = user_table.shape
    idx2d = inputs.astype(jnp.int32).reshape(batch // _IDXROW, _IDXROW)
    gather = _make_gather(num_rows, embed_dim, batch)
    out_t = gather(idx2d, user_table.T)
    return out_t.T
